# Initial kernel scaffold; baseline (speedup 1.0000x reference)
import jax
import jax.numpy as jnp
from jax.experimental import pallas as pl


def kernel(src, table, W_ih, W_hh, b_ih, b_hh):
    raise NotImplementedError("write your pallas kernel here")


# speedup vs baseline: 3.8498x; 3.8498x over previous
import functools

import jax
import jax.numpy as jnp
from jax import lax
from jax.experimental import pallas as pl
from jax.experimental.pallas import tpu as pltpu
from jax.experimental.pallas import tpu_sc as plsc

_V = 1000000
_E = 64
_H = 128
_B = 1024
_T = 200

_NW = 32
_CHUNK = 128
_FIRE = 10
_WAVES = 5


def _sc_gather_body(table_hbm, idx_hbm, out_hbm, idx_v, rows_v, sem):
    wid = lax.axis_index("s") * 2 + lax.axis_index("c")

    def wave(wv, carry):
        blk = wid * _WAVES + wv
        pltpu.sync_copy(idx_hbm.at[blk], idx_v)
        copies = [
            pltpu.async_copy(
                table_hbm.at[idx_v.at[j, 0]],
                rows_v.at[pl.ds(j * _CHUNK, _CHUNK)],
                sem,
            )
            for j in range(_FIRE)
        ]
        for cp in copies:
            cp.wait()
        pltpu.sync_copy(
            rows_v, out_hbm.at[pl.ds(blk * _FIRE * _CHUNK, _FIRE * _CHUNK)])
        return carry

    lax.fori_loop(0, _WAVES, wave, 0)


@functools.lru_cache(maxsize=1)
def _sc_gather_fn():
    return pl.kernel(
        _sc_gather_body,
        mesh=plsc.VectorSubcoreMesh(core_axis_name="c", subcore_axis_name="s"),
        out_type=jax.ShapeDtypeStruct((_T * _B, _E), jnp.float32),
        scratch_types=[
            pltpu.VMEM((_FIRE, 1, _CHUNK), jnp.int32),
            pltpu.VMEM((_FIRE * _CHUNK, _E), jnp.float32),
            pltpu.SemaphoreType.DMA,
        ],
        compiler_params=pltpu.CompilerParams(use_tc_tiling_on_sc=False),
    )


def _gru_step(x_ref, wih_ref, whh_ref, bi_ref, bh_ref, out_ref, h_ref):
    t = pl.program_id(0)

    @pl.when(t == 0)
    def _init():
        h_ref[...] = jnp.zeros_like(h_ref)

    x = x_ref[0]
    h = h_ref[...]
    dn = (((1,), (0,)), ((), ()))
    gi = lax.dot_general(x, wih_ref[...], dn,
                         preferred_element_type=jnp.float32) + bi_ref[...]
    gh = lax.dot_general(h, whh_ref[...], dn,
                         preferred_element_type=jnp.float32) + bh_ref[...]
    r = jax.nn.sigmoid(gi[:, :_H] + gh[:, :_H])
    z = jax.nn.sigmoid(gi[:, _H:2 * _H] + gh[:, _H:2 * _H])
    n = jnp.tanh(gi[:, 2 * _H:] + r * gh[:, 2 * _H:])
    h_new = (1.0 - z) * n + z * h
    h_ref[...] = h_new
    out_ref[0] = h_new


def _gru_scan(emb_tbe, wih_t, whh_t, bi2, bh2, interpret=False):
    return pl.pallas_call(
        _gru_step,
        grid=(_T,),
        in_specs=[
            pl.BlockSpec((1, _B, _E), lambda t: (t, 0, 0)),
            pl.BlockSpec((_E, 3 * _H), lambda t: (0, 0)),
            pl.BlockSpec((_H, 3 * _H), lambda t: (0, 0)),
            pl.BlockSpec((1, 3 * _H), lambda t: (0, 0)),
            pl.BlockSpec((1, 3 * _H), lambda t: (0, 0)),
        ],
        out_specs=pl.BlockSpec((1, _B, _H), lambda t: (t, 0, 0)),
        out_shape=jax.ShapeDtypeStruct((_T, _B, _H), jnp.float32),
        scratch_shapes=[pltpu.VMEM((_B, _H), jnp.float32)],
        interpret=interpret,
    )(emb_tbe, wih_t, whh_t, bi2, bh2)


def kernel(src, table, W_ih, W_hh, b_ih, b_hh):
    idx = jnp.swapaxes(src, 0, 1).reshape(_NW * _WAVES, _FIRE, 1, _CHUNK)
    emb = _sc_gather_fn()(table, idx)
    emb_tbe = emb.reshape(_T, _B, _E)

    outs = _gru_scan(
        emb_tbe,
        W_ih.T,
        W_hh.T,
        b_ih.reshape(1, 3 * _H),
        b_hh.reshape(1, 3 * _H),
    )

    outputs = jnp.swapaxes(outs, 0, 1)
    hidden = outs[_T - 1][None]
    return (outputs, hidden)


# speedup vs baseline: 4.1798x; 1.0857x over previous
import functools

import jax
import jax.numpy as jnp
from jax import lax
from jax.experimental import pallas as pl
from jax.experimental.pallas import tpu as pltpu
from jax.experimental.pallas import tpu_sc as plsc

_V = 1000000
_E = 64
_H = 128
_B = 1024
_T = 200

_NW = 32
_CHUNK = 128
_FIRE = 10
_WAVES = 5


def _sc_gather_body(table_hbm, idx_hbm, out_hbm, idx_v, rows_v, sem):
    wid = lax.axis_index("s") * 2 + lax.axis_index("c")

    def wave(wv, carry):
        blk = wid * _WAVES + wv
        pltpu.sync_copy(idx_hbm.at[blk], idx_v)
        copies = [
            pltpu.async_copy(
                table_hbm.at[idx_v.at[j, 0]],
                rows_v.at[pl.ds(j * _CHUNK, _CHUNK)],
                sem,
            )
            for j in range(_FIRE)
        ]
        for cp in copies:
            cp.wait()
        pltpu.sync_copy(
            rows_v, out_hbm.at[pl.ds(blk * _FIRE * _CHUNK, _FIRE * _CHUNK)])
        return carry

    lax.fori_loop(0, _WAVES, wave, 0)


@functools.lru_cache(maxsize=1)
def _sc_gather_fn():
    return pl.kernel(
        _sc_gather_body,
        mesh=plsc.VectorSubcoreMesh(core_axis_name="c", subcore_axis_name="s"),
        out_type=jax.ShapeDtypeStruct((_T * _B, _E), jnp.float32),
        scratch_types=[
            pltpu.VMEM((_FIRE, 1, _CHUNK), jnp.int32),
            pltpu.VMEM((_FIRE * _CHUNK, _E), jnp.float32),
            pltpu.SemaphoreType.DMA,
        ],
        compiler_params=pltpu.CompilerParams(use_tc_tiling_on_sc=False),
    )


_TS = 8
_NB = _T // _TS


def _gru_step(x_ref, wih_ref, whh_ref, bi_ref, bh_ref,
              out_ref, hid_ref, h_ref):
    k = pl.program_id(0)

    @pl.when(k == 0)
    def _init():
        h_ref[...] = jnp.zeros_like(h_ref)

    dn = (((1,), (0,)), ((), ()))
    gi_all = lax.dot_general(x_ref[...], wih_ref[...], dn,
                             preferred_element_type=jnp.float32) + bi_ref[...]
    h = h_ref[...]
    for i in range(_TS):
        gi = gi_all[i * _B:(i + 1) * _B]
        gh = lax.dot_general(h, whh_ref[...], dn,
                             preferred_element_type=jnp.float32) + bh_ref[...]
        r = jax.nn.sigmoid(gi[:, :_H] + gh[:, :_H])
        z = jax.nn.sigmoid(gi[:, _H:2 * _H] + gh[:, _H:2 * _H])
        n = jnp.tanh(gi[:, 2 * _H:] + r * gh[:, 2 * _H:])
        h = (1.0 - z) * n + z * h
        out_ref[:, i, :] = h
    h_ref[...] = h

    @pl.when(k == _NB - 1)
    def _final():
        hid_ref[0] = h


def _gru_scan(emb_2d, wih_t, whh_t, bi2, bh2, interpret=False):
    return pl.pallas_call(
        _gru_step,
        grid=(_NB,),
        in_specs=[
            pl.BlockSpec((_TS * _B, _E), lambda k: (k, 0)),
            pl.BlockSpec((_E, 3 * _H), lambda k: (0, 0)),
            pl.BlockSpec((_H, 3 * _H), lambda k: (0, 0)),
            pl.BlockSpec((1, 3 * _H), lambda k: (0, 0)),
            pl.BlockSpec((1, 3 * _H), lambda k: (0, 0)),
        ],
        out_specs=[
            pl.BlockSpec((_B, _TS, _H), lambda k: (0, k, 0)),
            pl.BlockSpec((1, _B, _H), lambda k: (0, 0, 0)),
        ],
        out_shape=[
            jax.ShapeDtypeStruct((_B, _T, _H), jnp.float32),
            jax.ShapeDtypeStruct((1, _B, _H), jnp.float32),
        ],
        scratch_shapes=[pltpu.VMEM((_B, _H), jnp.float32)],
        interpret=interpret,
    )(emb_2d, wih_t, whh_t, bi2, bh2)


def kernel(src, table, W_ih, W_hh, b_ih, b_hh):
    idx = jnp.swapaxes(src, 0, 1).reshape(_NW * _WAVES, _FIRE, 1, _CHUNK)
    emb = _sc_gather_fn()(table, idx)

    outputs, hidden = _gru_scan(
        emb,
        W_ih.T,
        W_hh.T,
        b_ih.reshape(1, 3 * _H),
        b_hh.reshape(1, 3 * _H),
    )
    return (outputs, hidden)


# speedup vs baseline: 4.1821x; 1.0005x over previous
import functools

import jax
import jax.numpy as jnp
from jax import lax
from jax.experimental import pallas as pl
from jax.experimental.pallas import tpu as pltpu
from jax.experimental.pallas import tpu_sc as plsc

_V = 1000000
_E = 64
_H = 128
_B = 1024
_T = 200

_NW = 32
_CHUNK = 128
_FIRE = 10
_WAVES = 5


def _sc_gather_body(table_hbm, idx_hbm, out_hbm, idx_v, rows_v, sem):
    wid = lax.axis_index("s") * 2 + lax.axis_index("c")

    def wave(wv, carry):
        blk = wid * _WAVES + wv
        pltpu.sync_copy(idx_hbm.at[blk], idx_v)
        copies = [
            pltpu.async_copy(
                table_hbm.at[idx_v.at[j, 0]],
                rows_v.at[pl.ds(j * _CHUNK, _CHUNK)],
                sem,
            )
            for j in range(_FIRE)
        ]
        for cp in copies:
            cp.wait()
        nrow = _FIRE * _CHUNK
        pltpu.sync_copy(rows_v, out_hbm.at[pl.ds(blk * nrow, nrow)])
        return carry

    lax.fori_loop(0, _WAVES, wave, 0)


@functools.lru_cache(maxsize=1)
def _sc_gather_fn():
    return pl.kernel(
        _sc_gather_body,
        mesh=plsc.VectorSubcoreMesh(core_axis_name="c", subcore_axis_name="s"),
        out_type=jax.ShapeDtypeStruct((_T * _B, _E), jnp.float32),
        scratch_types=[
            pltpu.VMEM((_FIRE, 1, _CHUNK), jnp.int32),
            pltpu.VMEM((_FIRE * _CHUNK, _E), jnp.float32),
            pltpu.SemaphoreType.DMA,
        ],
        compiler_params=pltpu.CompilerParams(use_tc_tiling_on_sc=False),
    )


_TS = 8
_NB = _T // _TS


def _gru_step(x_ref, wih_ref, whh_ref, bi_ref, bh_ref,
              out_ref, hid_ref, h_ref):
    k = pl.program_id(0)

    @pl.when(k == 0)
    def _init():
        h_ref[...] = jnp.zeros_like(h_ref)

    dn = (((1,), (0,)), ((), ()))
    gi = lax.dot_general(x_ref[...], wih_ref[...], dn,
                         preferred_element_type=jnp.float32) + bi_ref[...]
    h = h_ref[...]
    for i in range(_TS):
        gh = lax.dot_general(h, whh_ref[...], dn,
                             preferred_element_type=jnp.float32) + bh_ref[...]
        g = gi[i * _B:(i + 1) * _B]
        r = jax.nn.sigmoid(g[:, :_H] + gh[:, :_H])
        z = jax.nn.sigmoid(g[:, _H:2 * _H] + gh[:, _H:2 * _H])
        n = jnp.tanh(g[:, 2 * _H:] + r * gh[:, 2 * _H:])
        h = (1.0 - z) * n + z * h
        out_ref[:, i, :] = h
    h_ref[...] = h

    @pl.when(k == _NB - 1)
    def _final():
        hid_ref[0] = h


def _gru_scan(emb_2d, wih_t, whh_t, bi2, bh2, interpret=False):
    return pl.pallas_call(
        _gru_step,
        grid=(_NB,),
        in_specs=[
            pl.BlockSpec((_TS * _B, _E), lambda k: (k, 0)),
            pl.BlockSpec((_E, 3 * _H), lambda k: (0, 0)),
            pl.BlockSpec((_H, 3 * _H), lambda k: (0, 0)),
            pl.BlockSpec((1, 3 * _H), lambda k: (0, 0)),
            pl.BlockSpec((1, 3 * _H), lambda k: (0, 0)),
        ],
        out_specs=[
            pl.BlockSpec((_B, _TS, _H), lambda k: (0, k, 0)),
            pl.BlockSpec((1, _B, _H), lambda k: (0, 0, 0)),
        ],
        out_shape=[
            jax.ShapeDtypeStruct((_B, _T, _H), jnp.float32),
            jax.ShapeDtypeStruct((1, _B, _H), jnp.float32),
        ],
        scratch_shapes=[pltpu.VMEM((_B, _H), jnp.float32)],
        interpret=interpret,
    )(emb_2d, wih_t, whh_t, bi2, bh2)


def kernel(src, table, W_ih, W_hh, b_ih, b_hh):
    idx = (jnp.swapaxes(src, 0, 1)
           .reshape(_NW * _WAVES, _FIRE, 1, _CHUNK))
    emb2 = _sc_gather_fn()(table, idx)

    outputs, hidden = _gru_scan(
        emb2,
        W_ih.T,
        W_hh.T,
        b_ih.reshape(1, 3 * _H),
        b_hh.reshape(1, 3 * _H),
    )
    return (outputs, hidden)


# speedup vs baseline: 4.2221x; 1.0096x over previous
import functools

import jax
import jax.numpy as jnp
from jax import lax
from jax.experimental import pallas as pl
from jax.experimental.pallas import tpu as pltpu
from jax.experimental.pallas import tpu_sc as plsc

_V = 1000000
_E = 64
_H = 128
_B = 1024
_T = 200

_NW = 32
_CHUNK = 128
_FIRE = 10
_WAVES = 5


def _sc_gather_body(table_hbm, idx_hbm, out_hbm, idx_v, rows_v, sem):
    wid = lax.axis_index("s") * 2 + lax.axis_index("c")

    def wave(wv, carry):
        blk = wid * _WAVES + wv
        pltpu.sync_copy(idx_hbm.at[blk], idx_v)
        copies = [
            pltpu.async_copy(
                table_hbm.at[idx_v.at[j, 0]],
                rows_v.at[pl.ds(j * _CHUNK, _CHUNK)],
                sem,
            )
            for j in range(_FIRE)
        ]
        for cp in copies:
            cp.wait()
        nrow = _FIRE * _CHUNK
        pltpu.sync_copy(rows_v, out_hbm.at[pl.ds(blk * nrow, nrow)])
        return carry

    lax.fori_loop(0, _WAVES, wave, 0)


@functools.lru_cache(maxsize=1)
def _sc_gather_fn():
    return pl.kernel(
        _sc_gather_body,
        mesh=plsc.VectorSubcoreMesh(core_axis_name="c", subcore_axis_name="s"),
        out_type=jax.ShapeDtypeStruct((_T * _B, _E), jnp.float32),
        scratch_types=[
            pltpu.VMEM((_FIRE, 1, _CHUNK), jnp.int32),
            pltpu.VMEM((_FIRE * _CHUNK, _E), jnp.float32),
            pltpu.SemaphoreType.DMA,
        ],
        compiler_params=pltpu.CompilerParams(use_tc_tiling_on_sc=False),
    )


_TS = 8
_NB = _T // _TS


def _gru_step(x_ref, wih_ref, whh_ref, bi_ref, bh_ref,
              out_ref, hid_ref, h_ref):
    k = pl.program_id(0)

    @pl.when(k == 0)
    def _init():
        h_ref[...] = jnp.zeros_like(h_ref)

    dn = (((1,), (0,)), ((), ()))
    h = h_ref[...]
    for i in range(_TS):
        gi = lax.dot_general(x_ref[:, i, :], wih_ref[...], dn,
                             preferred_element_type=jnp.float32) + bi_ref[...]
        gh = lax.dot_general(h, whh_ref[...], dn,
                             preferred_element_type=jnp.float32) + bh_ref[...]
        g = gi
        r = jax.nn.sigmoid(g[:, :_H] + gh[:, :_H])
        z = jax.nn.sigmoid(g[:, _H:2 * _H] + gh[:, _H:2 * _H])
        n = jnp.tanh(g[:, 2 * _H:] + r * gh[:, 2 * _H:])
        h = (1.0 - z) * n + z * h
        out_ref[:, i, :] = h
    h_ref[...] = h

    @pl.when(k == _NB - 1)
    def _final():
        hid_ref[0] = h


def _gru_scan(emb_2d, wih_t, whh_t, bi2, bh2, interpret=False):
    return pl.pallas_call(
        _gru_step,
        grid=(_NB,),
        in_specs=[
            pl.BlockSpec((_B, _TS, _E), lambda k: (0, k, 0)),
            pl.BlockSpec((_E, 3 * _H), lambda k: (0, 0)),
            pl.BlockSpec((_H, 3 * _H), lambda k: (0, 0)),
            pl.BlockSpec((1, 3 * _H), lambda k: (0, 0)),
            pl.BlockSpec((1, 3 * _H), lambda k: (0, 0)),
        ],
        out_specs=[
            pl.BlockSpec((_B, _TS, _H), lambda k: (0, k, 0)),
            pl.BlockSpec((1, _B, _H), lambda k: (0, 0, 0)),
        ],
        out_shape=[
            jax.ShapeDtypeStruct((_B, _T, _H), jnp.float32),
            jax.ShapeDtypeStruct((1, _B, _H), jnp.float32),
        ],
        scratch_shapes=[pltpu.VMEM((_B, _H), jnp.float32)],
        interpret=interpret,
    )(emb_2d, wih_t, whh_t, bi2, bh2)


def kernel(src, table, W_ih, W_hh, b_ih, b_hh):
    idx = src.reshape(_NW * _WAVES, _FIRE, 1, _CHUNK)
    emb2 = _sc_gather_fn()(table, idx)

    outputs, hidden = _gru_scan(
        emb2.reshape(_B, _T, _E),
        W_ih.T,
        W_hh.T,
        b_ih.reshape(1, 3 * _H),
        b_hh.reshape(1, 3 * _H),
    )
    return (outputs, hidden)


# speedup vs baseline: 5.8766x; 1.3919x over previous
import functools

import jax
import jax.numpy as jnp
from jax import lax
from jax.experimental.layout import Format, Layout, with_layout_constraint
from jax.experimental import pallas as pl
from jax.experimental.pallas import tpu as pltpu
from jax.experimental.pallas import tpu_sc as plsc

_V = 1000000
_E = 64
_H = 128
_B = 1024
_T = 200

_NW = 32
_CHUNK = 128
_FIRE = 10
_WAVES = 5


def _sc_gather_body(table_hbm, idx_hbm, out_hbm, idx_v, rows_v, sem):
    wid = lax.axis_index("s") * 2 + lax.axis_index("c")

    def wave(wv, carry):
        blk = wid * _WAVES + wv
        pltpu.sync_copy(idx_hbm.at[blk], idx_v)
        copies = [
            pltpu.async_copy(
                table_hbm.at[idx_v.at[j, 0]],
                rows_v.at[pl.ds(j * _CHUNK, _CHUNK)],
                sem,
            )
            for j in range(_FIRE)
        ]
        for cp in copies:
            cp.wait()
        nrow = _FIRE * _CHUNK
        pltpu.sync_copy(rows_v, out_hbm.at[pl.ds(blk * nrow, nrow)])
        return carry

    lax.fori_loop(0, _WAVES, wave, 0)


@functools.lru_cache(maxsize=1)
def _sc_gather_fn():
    return pl.kernel(
        _sc_gather_body,
        mesh=plsc.VectorSubcoreMesh(core_axis_name="c", subcore_axis_name="s"),
        out_type=jax.ShapeDtypeStruct((_T * _B, _E), jnp.float32),
        scratch_types=[
            pltpu.VMEM((_FIRE, 1, _CHUNK), jnp.int32),
            pltpu.VMEM((_FIRE * _CHUNK, _E), jnp.float32),
            pltpu.SemaphoreType.DMA,
        ],
        compiler_params=pltpu.CompilerParams(use_tc_tiling_on_sc=False),
    )


_TS = 8
_NB = _T // _TS


def _gru_step(x_ref, wih_ref, whh_ref, bi_ref, bh_ref,
              out_ref, hid_ref, h_ref):
    k = pl.program_id(0)

    @pl.when(k == 0)
    def _init():
        h_ref[...] = jnp.zeros_like(h_ref)

    dn = (((1,), (0,)), ((), ()))
    h = h_ref[...]
    for i in range(_TS):
        gi = lax.dot_general(x_ref[:, i, :], wih_ref[...], dn,
                             preferred_element_type=jnp.float32) + bi_ref[...]
        gh = lax.dot_general(h, whh_ref[...], dn,
                             preferred_element_type=jnp.float32) + bh_ref[...]
        g = gi
        r = jax.nn.sigmoid(g[:, :_H] + gh[:, :_H])
        z = jax.nn.sigmoid(g[:, _H:2 * _H] + gh[:, _H:2 * _H])
        n = jnp.tanh(g[:, 2 * _H:] + r * gh[:, 2 * _H:])
        h = (1.0 - z) * n + z * h
        out_ref[:, i, :] = h
    h_ref[...] = h

    @pl.when(k == _NB - 1)
    def _final():
        hid_ref[0] = h


def _gru_scan(emb_2d, wih_t, whh_t, bi2, bh2, interpret=False):
    return pl.pallas_call(
        _gru_step,
        grid=(_NB,),
        in_specs=[
            pl.BlockSpec((_B, _TS, _E), lambda k: (0, k, 0)),
            pl.BlockSpec((_E, 3 * _H), lambda k: (0, 0)),
            pl.BlockSpec((_H, 3 * _H), lambda k: (0, 0)),
            pl.BlockSpec((1, 3 * _H), lambda k: (0, 0)),
            pl.BlockSpec((1, 3 * _H), lambda k: (0, 0)),
        ],
        out_specs=[
            pl.BlockSpec((_B, _TS, _H), lambda k: (0, k, 0)),
            pl.BlockSpec((1, _B, _H), lambda k: (0, 0, 0)),
        ],
        out_shape=[
            jax.ShapeDtypeStruct((_B, _T, _H), jnp.float32),
            jax.ShapeDtypeStruct((1, _B, _H), jnp.float32),
        ],
        scratch_shapes=[pltpu.VMEM((_B, _H), jnp.float32)],
        interpret=interpret,
    )(emb_2d, wih_t, whh_t, bi2, bh2)


def kernel(src, table, W_ih, W_hh, b_ih, b_hh):
    idx = src.reshape(_NW * _WAVES, _FIRE, 1, _CHUNK)
    table_l = with_layout_constraint(
        table, Layout(major_to_minor=(0, 1), tiling=((8,),)))
    emb2 = _sc_gather_fn()(table_l, idx)

    outputs, hidden = _gru_scan(
        emb2.reshape(_B, _T, _E),
        W_ih.T,
        W_hh.T,
        b_ih.reshape(1, 3 * _H),
        b_hh.reshape(1, 3 * _H),
    )
    return (outputs, hidden)
